# baseline (speedup 1.0000x reference)
import functools

import jax
import jax.numpy as jnp
from jax import lax
from jax.experimental import pallas as pl
from jax.experimental.pallas import tpu as pltpu
from jax.experimental.pallas import tpu_sc as plsc

_NT = 512


def _cdist_body(src_ref, dst_ref, dist_f_ref, idx_f_ref, dist_b_ref,
                idx_b_ref, row_min_ref, row_idx_ref):
    j = pl.program_id(1)
    nj = pl.num_programs(1)
    xs = src_ref[0]
    ys = dst_ref[0]
    m = xs.shape[0]
    nt = ys.shape[1]

    g = lax.dot_general(xs, ys, (((1,), (0,)), ((), ())),
                        preferred_element_type=jnp.float32,
                        precision=lax.Precision.HIGHEST)
    xx = jnp.sum(xs * xs, axis=1, keepdims=True)
    yy = jnp.sum(ys * ys, axis=0, keepdims=True)
    d2 = jnp.maximum(xx + yy - 2.0 * g, 0.0)

    col_iota = lax.broadcasted_iota(jnp.int32, (m, nt), 1)
    row_iota = lax.broadcasted_iota(jnp.int32, (m, nt), 0)

    bb = pl.program_id(0)
    n_total = nj * nt

    tmin = jnp.min(d2, axis=1, keepdims=True)
    tidx = (jnp.min(jnp.where(d2 == tmin, col_iota, nt),
                    axis=1, keepdims=True) + j * nt + bb * n_total)

    @pl.when(j == 0)
    def _():
        row_min_ref[...] = tmin
        row_idx_ref[...] = tidx

    @pl.when(j > 0)
    def _():
        better = tmin < row_min_ref[...]
        row_min_ref[...] = jnp.where(better, tmin, row_min_ref[...])
        row_idx_ref[...] = jnp.where(better, tidx, row_idx_ref[...])

    cmin = jnp.min(d2, axis=0, keepdims=True)
    cidx = jnp.min(jnp.where(d2 == cmin, row_iota, m),
                   axis=0, keepdims=True) + bb * m
    dist_b_ref[...] = jnp.sqrt(cmin)[None]
    idx_b_ref[...] = cidx[None]

    @pl.when(j == nj - 1)
    def _():
        dist_f_ref[...] = jnp.sqrt(row_min_ref[...])[None]
        idx_f_ref[...] = row_idx_ref[...][None]


def _cdist_call(src_t, dst, *, interpret=False):
    b, m, _ = src_t.shape
    n = dst.shape[2]
    nj = n // _NT
    return pl.pallas_call(
        _cdist_body,
        grid=(b, nj),
        in_specs=[
            pl.BlockSpec((1, m, 3), lambda b_, j: (b_, 0, 0)),
            pl.BlockSpec((1, 3, _NT), lambda b_, j: (b_, 0, j)),
        ],
        out_specs=[
            pl.BlockSpec((1, m, 1), lambda b_, j: (b_, 0, 0)),
            pl.BlockSpec((1, m, 1), lambda b_, j: (b_, 0, 0)),
            pl.BlockSpec((1, 1, _NT), lambda b_, j: (b_, 0, j)),
            pl.BlockSpec((1, 1, _NT), lambda b_, j: (b_, 0, j)),
        ],
        out_shape=[
            jax.ShapeDtypeStruct((b, m, 1), jnp.float32),
            jax.ShapeDtypeStruct((b, m, 1), jnp.int32),
            jax.ShapeDtypeStruct((b, 1, n), jnp.float32),
            jax.ShapeDtypeStruct((b, 1, n), jnp.int32),
        ],
        scratch_shapes=[
            pltpu.VMEM((m, 1), jnp.float32),
            pltpu.VMEM((m, 1), jnp.int32),
        ],
        interpret=interpret,
    )(src_t, dst)


_IW = 128


def _make_sc_gather(b, m, n):
    info = plsc.get_sparse_core_info()
    nc, ns, lanes = info.num_cores, info.num_subcores, info.num_lanes
    nw = nc * ns
    fw = (b * m) // nw
    bw = (b * n) // nw
    assert (b * m) % (nw * _IW) == 0 and (b * n) % (nw * _IW) == 0
    kf = fw // _IW
    kb = bw // _IW
    f_scale = 0.5 / (b * m)
    b_scale = 0.5 / (b * n)
    mesh = plsc.VectorSubcoreMesh(core_axis_name="c", subcore_axis_name="s")

    @functools.partial(
        pl.kernel, mesh=mesh,
        out_type=jax.ShapeDtypeStruct((nw, lanes), jnp.float32),
        scratch_types=[
            pltpu.VMEM((kf, _IW), jnp.int32),
            pltpu.VMEM((kf, _IW), jnp.float32),
            pltpu.VMEM((fw,), jnp.float32),
            pltpu.VMEM((fw,), jnp.float32),
            pltpu.VMEM((kb, _IW), jnp.int32),
            pltpu.VMEM((kb, _IW), jnp.float32),
            pltpu.VMEM((bw,), jnp.float32),
            pltpu.VMEM((bw,), jnp.float32),
            pltpu.VMEM((lanes,), jnp.float32),
            pltpu.SemaphoreType.DMA,
        ],
    )
    def sc_fn(df_hbm, if_hbm, ss_hbm, db_hbm, ib_hbm, sd_hbm, out_hbm,
              fidx_v, fgth_v, fdat_v, fsig_v,
              bidx_v, bgth_v, bdat_v, bsig_v, acc_v, sem):
        wid = lax.axis_index("s") * nc + lax.axis_index("c")

        fbase = wid * fw
        pltpu.sync_copy(if_hbm.at[pl.ds(wid * kf, kf)], fidx_v)
        pltpu.sync_copy(df_hbm.at[pl.ds(fbase, fw)], fdat_v)
        pltpu.sync_copy(ss_hbm.at[pl.ds(fbase, fw)], fsig_v)
        for k in range(kf):
            pltpu.async_copy(sd_hbm.at[fidx_v.at[k]], fgth_v.at[k], sem).wait()

        bbase = wid * bw
        pltpu.sync_copy(ib_hbm.at[pl.ds(wid * kb, kb)], bidx_v)
        pltpu.sync_copy(db_hbm.at[pl.ds(bbase, bw)], bdat_v)
        pltpu.sync_copy(sd_hbm.at[pl.ds(bbase, bw)], bsig_v)
        for k in range(kb):
            pltpu.async_copy(ss_hbm.at[bidx_v.at[k]], bgth_v.at[k], sem).wait()

        acc_f = jnp.zeros((lanes,), jnp.float32)
        for c in range(fw // lanes):
            k, off = divmod(c * lanes, _IW)
            gth = fgth_v[k, pl.ds(off, lanes)]
            sl = pl.ds(c * lanes, lanes)
            acc_f = acc_f + fdat_v[sl] * (fsig_v[sl] + gth)

        acc_b = jnp.zeros((lanes,), jnp.float32)
        for c in range(bw // lanes):
            k, off = divmod(c * lanes, _IW)
            gth = bgth_v[k, pl.ds(off, lanes)]
            sl = pl.ds(c * lanes, lanes)
            acc_b = acc_b + bdat_v[sl] * (bsig_v[sl] + gth)

        acc_v[...] = acc_f * f_scale + acc_b * b_scale
        pltpu.sync_copy(acc_v, out_hbm.at[wid])

    return sc_fn


def _finalize_body(p_ref, o_ref):
    o_ref[...] = jnp.full((1, 1), jnp.sum(p_ref[...]), jnp.float32)


def _finalize(parts, *, interpret=False):
    return pl.pallas_call(
        _finalize_body,
        out_shape=jax.ShapeDtypeStruct((1, 1), jnp.float32),
        interpret=interpret,
    )(parts)


def kernel(pc_src, pc_dst, sigma_src, sigma_dst):
    b, _, m = pc_src.shape
    n = pc_dst.shape[2]
    src_t = jnp.transpose(pc_src, (0, 2, 1))
    dist_f, idx_f, dist_b, idx_b = _cdist_call(src_t, pc_dst)
    sc_fn = _make_sc_gather(b, m, n)
    parts = sc_fn(dist_f.reshape(-1), idx_f.reshape(-1, _IW),
                  sigma_src.reshape(-1), dist_b.reshape(-1),
                  idx_b.reshape(-1, _IW), sigma_dst.reshape(-1))
    return _finalize(parts)[0, 0]


# speedup vs baseline: 1.2578x; 1.2578x over previous
import functools

import jax
import jax.numpy as jnp
from jax import lax
from jax.experimental import pallas as pl
from jax.experimental.pallas import tpu as pltpu
from jax.experimental.pallas import tpu_sc as plsc

_NT = 2048


def _cdist_body(src2_ref, dst_ref, dist_f_ref, idx_f_ref, dist_b_ref,
                idx_b_ref, row_key_ref):
    j = pl.program_id(1)
    nj = pl.num_programs(1)
    bb = pl.program_id(0)
    xs2 = src2_ref[0]
    ys = dst_ref[0]
    m = xs2.shape[0]
    nt = ys.shape[1]
    n_total = nj * nt

    g2 = lax.dot_general(xs2, ys, (((1,), (0,)), ((), ())),
                         preferred_element_type=jnp.float32,
                         precision=lax.Precision.HIGHEST)
    xx = 0.25 * jnp.sum(xs2 * xs2, axis=1, keepdims=True)
    yy = jnp.sum(ys * ys, axis=0, keepdims=True)
    s = (xx + yy) - g2
    bits = lax.bitcast_convert_type(s, jnp.int32)
    km = jnp.maximum(bits, 0) & jnp.int32(-2048)

    iota_c = lax.broadcasted_iota(jnp.int32, (m, nt), 1)
    tkey = jnp.min(km | iota_c, axis=1, keepdims=True)
    gkey = tkey + j * nt

    @pl.when(j == 0)
    def _():
        row_key_ref[...] = gkey

    @pl.when(j > 0)
    def _():
        row_key_ref[...] = jnp.minimum(row_key_ref[...], gkey)

    iota_r = lax.broadcasted_iota(jnp.int32, (m, nt), 0)
    ckey = jnp.min(km | iota_r, axis=0, keepdims=True)
    cval = lax.bitcast_convert_type(ckey & jnp.int32(-2048), jnp.float32)
    dist_b_ref[...] = jnp.sqrt(cval)[None]
    idx_b_ref[...] = ((ckey & 0x7FF) + bb * m)[None]

    @pl.when(j == nj - 1)
    def _():
        rk = row_key_ref[...]
        rval = lax.bitcast_convert_type(rk & jnp.int32(-2048), jnp.float32)
        dist_f_ref[...] = jnp.sqrt(rval)[None]
        idx_f_ref[...] = ((rk & 0x7FF) + bb * n_total)[None]


def _cdist_call(src2, dst, *, interpret=False):
    b, m, _ = src2.shape
    n = dst.shape[2]
    nj = n // _NT
    assert n <= 2048 and m <= 2048
    return pl.pallas_call(
        _cdist_body,
        grid=(b, nj),
        in_specs=[
            pl.BlockSpec((1, m, 3), lambda b_, j: (b_, 0, 0)),
            pl.BlockSpec((1, 3, _NT), lambda b_, j: (b_, 0, j)),
        ],
        out_specs=[
            pl.BlockSpec((1, m, 1), lambda b_, j: (b_, 0, 0)),
            pl.BlockSpec((1, m, 1), lambda b_, j: (b_, 0, 0)),
            pl.BlockSpec((1, 1, _NT), lambda b_, j: (b_, 0, j)),
            pl.BlockSpec((1, 1, _NT), lambda b_, j: (b_, 0, j)),
        ],
        out_shape=[
            jax.ShapeDtypeStruct((b, m, 1), jnp.float32),
            jax.ShapeDtypeStruct((b, m, 1), jnp.int32),
            jax.ShapeDtypeStruct((b, 1, n), jnp.float32),
            jax.ShapeDtypeStruct((b, 1, n), jnp.int32),
        ],
        scratch_shapes=[
            pltpu.VMEM((m, 1), jnp.int32),
        ],
        interpret=interpret,
    )(src2, dst)


_IW = 128


def _make_sc_gather(b, m, n):
    info = plsc.get_sparse_core_info()
    nc, ns, lanes = info.num_cores, info.num_subcores, info.num_lanes
    nw = nc * ns
    fw = (b * m) // nw
    bw = (b * n) // nw
    assert (b * m) % (nw * _IW) == 0 and (b * n) % (nw * _IW) == 0
    kf = fw // _IW
    kb = bw // _IW
    f_scale = 0.5 / (b * m)
    b_scale = 0.5 / (b * n)
    mesh = plsc.VectorSubcoreMesh(core_axis_name="c", subcore_axis_name="s")

    @functools.partial(
        pl.kernel, mesh=mesh,
        out_type=jax.ShapeDtypeStruct((nw, lanes), jnp.float32),
        scratch_types=[
            pltpu.VMEM((kf, _IW), jnp.int32),
            pltpu.VMEM((kf, _IW), jnp.float32),
            pltpu.VMEM((fw,), jnp.float32),
            pltpu.VMEM((fw,), jnp.float32),
            pltpu.VMEM((kb, _IW), jnp.int32),
            pltpu.VMEM((kb, _IW), jnp.float32),
            pltpu.VMEM((bw,), jnp.float32),
            pltpu.VMEM((bw,), jnp.float32),
            pltpu.VMEM((lanes,), jnp.float32),
            pltpu.SemaphoreType.DMA,
        ],
    )
    def sc_fn(df_hbm, if_hbm, ss_hbm, db_hbm, ib_hbm, sd_hbm, out_hbm,
              fidx_v, fgth_v, fdat_v, fsig_v,
              bidx_v, bgth_v, bdat_v, bsig_v, acc_v, sem):
        wid = lax.axis_index("s") * nc + lax.axis_index("c")

        fbase = wid * fw
        pltpu.sync_copy(if_hbm.at[pl.ds(wid * kf, kf)], fidx_v)
        pltpu.sync_copy(df_hbm.at[pl.ds(fbase, fw)], fdat_v)
        pltpu.sync_copy(ss_hbm.at[pl.ds(fbase, fw)], fsig_v)
        for k in range(kf):
            pltpu.async_copy(sd_hbm.at[fidx_v.at[k]], fgth_v.at[k], sem).wait()

        bbase = wid * bw
        pltpu.sync_copy(ib_hbm.at[pl.ds(wid * kb, kb)], bidx_v)
        pltpu.sync_copy(db_hbm.at[pl.ds(bbase, bw)], bdat_v)
        pltpu.sync_copy(sd_hbm.at[pl.ds(bbase, bw)], bsig_v)
        for k in range(kb):
            pltpu.async_copy(ss_hbm.at[bidx_v.at[k]], bgth_v.at[k], sem).wait()

        acc_f = jnp.zeros((lanes,), jnp.float32)
        for c in range(fw // lanes):
            k, off = divmod(c * lanes, _IW)
            gth = fgth_v[k, pl.ds(off, lanes)]
            sl = pl.ds(c * lanes, lanes)
            acc_f = acc_f + fdat_v[sl] * (fsig_v[sl] + gth)

        acc_b = jnp.zeros((lanes,), jnp.float32)
        for c in range(bw // lanes):
            k, off = divmod(c * lanes, _IW)
            gth = bgth_v[k, pl.ds(off, lanes)]
            sl = pl.ds(c * lanes, lanes)
            acc_b = acc_b + bdat_v[sl] * (bsig_v[sl] + gth)

        acc_v[...] = acc_f * f_scale + acc_b * b_scale
        pltpu.sync_copy(acc_v, out_hbm.at[wid])

    return sc_fn


def _finalize_body(p_ref, o_ref):
    o_ref[...] = jnp.full((1, 1), jnp.sum(p_ref[...]), jnp.float32)


def _finalize(parts, *, interpret=False):
    return pl.pallas_call(
        _finalize_body,
        out_shape=jax.ShapeDtypeStruct((1, 1), jnp.float32),
        interpret=interpret,
    )(parts)


def kernel(pc_src, pc_dst, sigma_src, sigma_dst):
    b, _, m = pc_src.shape
    n = pc_dst.shape[2]
    src2 = jnp.transpose(pc_src, (0, 2, 1)) * 2.0
    dist_f, idx_f, dist_b, idx_b = _cdist_call(src2, pc_dst)
    sc_fn = _make_sc_gather(b, m, n)
    parts = sc_fn(dist_f.reshape(-1), idx_f.reshape(-1, _IW),
                  sigma_src.reshape(-1), dist_b.reshape(-1),
                  idx_b.reshape(-1, _IW), sigma_dst.reshape(-1))
    return _finalize(parts)[0, 0]


# speedup vs baseline: 2.0645x; 1.6413x over previous
import functools

import jax
import jax.numpy as jnp
from jax import lax
from jax.experimental import pallas as pl
from jax.experimental.pallas import tpu as pltpu
from jax.experimental.pallas import tpu_sc as plsc

_NT = 2048


def _cdist_body(src_ref, dst_ref, dist_f_ref, idx_f_ref, dist_b_ref,
                idx_b_ref, row_key_ref):
    j = pl.program_id(1)
    nj = pl.num_programs(1)
    bb = pl.program_id(0)
    xs = src_ref[0]
    ys = dst_ref[0]
    m = xs.shape[0]
    nt = ys.shape[1]
    n_total = nj * nt
    f32, i32 = jnp.float32, jnp.int32

    d0 = xs[:, 0:1] - ys[0:1, :]
    d1 = xs[:, 1:2] - ys[1:2, :]
    d2_ = xs[:, 2:3] - ys[2:3, :]
    s = d0 * d0 + d1 * d1 + d2_ * d2_

    bits = lax.bitcast_convert_type(s, i32)
    km = bits & i32(-2048)

    iota_c = lax.broadcasted_iota(i32, (m, nt), 1)
    keyr = lax.bitcast_convert_type(km | iota_c, f32)
    tkey = lax.bitcast_convert_type(
        jnp.min(keyr, axis=1, keepdims=True), i32)
    gkey = tkey + j * nt

    @pl.when(j == 0)
    def _():
        row_key_ref[...] = gkey

    @pl.when(j > 0)
    def _():
        row_key_ref[...] = jnp.minimum(row_key_ref[...], gkey)

    iota_r = lax.broadcasted_iota(i32, (m, nt), 0)
    keyc = lax.bitcast_convert_type(km | iota_r, f32)
    ckey = lax.bitcast_convert_type(
        jnp.min(keyc, axis=0, keepdims=True), i32)
    cval = lax.bitcast_convert_type(ckey & i32(-2048), f32)
    dist_b_ref[...] = jnp.sqrt(jnp.maximum(cval, 0.0))[None]
    idx_b_ref[...] = ((ckey & 0x7FF) + bb * m)[None]

    @pl.when(j == nj - 1)
    def _():
        rk = row_key_ref[...]
        rval = lax.bitcast_convert_type(rk & i32(-2048), f32)
        dist_f_ref[...] = jnp.sqrt(jnp.maximum(rval, 0.0))[None]
        idx_f_ref[...] = ((rk & 0x7FF) + bb * n_total)[None]


def _cdist_call(src_t, dst, *, interpret=False):
    b, m, k = src_t.shape
    n = dst.shape[2]
    nj = n // _NT
    assert n <= 2048 and m <= 2048
    return pl.pallas_call(
        _cdist_body,
        grid=(b, nj),
        in_specs=[
            pl.BlockSpec((1, m, k), lambda b_, j: (b_, 0, 0)),
            pl.BlockSpec((1, k, _NT), lambda b_, j: (b_, 0, j)),
        ],
        out_specs=[
            pl.BlockSpec((1, m, 1), lambda b_, j: (b_, 0, 0)),
            pl.BlockSpec((1, m, 1), lambda b_, j: (b_, 0, 0)),
            pl.BlockSpec((1, 1, _NT), lambda b_, j: (b_, 0, j)),
            pl.BlockSpec((1, 1, _NT), lambda b_, j: (b_, 0, j)),
        ],
        out_shape=[
            jax.ShapeDtypeStruct((b, m, 1), jnp.float32),
            jax.ShapeDtypeStruct((b, m, 1), jnp.int32),
            jax.ShapeDtypeStruct((b, 1, n), jnp.float32),
            jax.ShapeDtypeStruct((b, 1, n), jnp.int32),
        ],
        scratch_shapes=[
            pltpu.VMEM((m, 1), jnp.int32),
        ],
        interpret=interpret,
    )(src_t, dst)


_IW = 128


def _make_sc_gather(b, m, n):
    info = plsc.get_sparse_core_info()
    nc, ns, lanes = info.num_cores, info.num_subcores, info.num_lanes
    nw = nc * ns
    fw = (b * m) // nw
    bw = (b * n) // nw
    assert (b * m) % (nw * _IW) == 0 and (b * n) % (nw * _IW) == 0
    kf = fw // _IW
    kb = bw // _IW
    f_scale = 0.5 / (b * m)
    b_scale = 0.5 / (b * n)
    mesh = plsc.VectorSubcoreMesh(core_axis_name="c", subcore_axis_name="s")

    @functools.partial(
        pl.kernel, mesh=mesh,
        out_type=jax.ShapeDtypeStruct((nw, lanes), jnp.float32),
        scratch_types=[
            pltpu.VMEM((kf, _IW), jnp.int32),
            pltpu.VMEM((kf, _IW), jnp.float32),
            pltpu.VMEM((fw,), jnp.float32),
            pltpu.VMEM((fw,), jnp.float32),
            pltpu.VMEM((kb, _IW), jnp.int32),
            pltpu.VMEM((kb, _IW), jnp.float32),
            pltpu.VMEM((bw,), jnp.float32),
            pltpu.VMEM((bw,), jnp.float32),
            pltpu.VMEM((lanes,), jnp.float32),
            pltpu.SemaphoreType.DMA,
        ],
    )
    def sc_fn(df_hbm, if_hbm, ss_hbm, db_hbm, ib_hbm, sd_hbm, out_hbm,
              fidx_v, fgth_v, fdat_v, fsig_v,
              bidx_v, bgth_v, bdat_v, bsig_v, acc_v, sem):
        wid = lax.axis_index("s") * nc + lax.axis_index("c")

        fbase = wid * fw
        pltpu.sync_copy(if_hbm.at[pl.ds(wid * kf, kf)], fidx_v)
        pltpu.sync_copy(df_hbm.at[pl.ds(fbase, fw)], fdat_v)
        pltpu.sync_copy(ss_hbm.at[pl.ds(fbase, fw)], fsig_v)
        for k in range(kf):
            pltpu.async_copy(sd_hbm.at[fidx_v.at[k]], fgth_v.at[k], sem).wait()

        bbase = wid * bw
        pltpu.sync_copy(ib_hbm.at[pl.ds(wid * kb, kb)], bidx_v)
        pltpu.sync_copy(db_hbm.at[pl.ds(bbase, bw)], bdat_v)
        pltpu.sync_copy(sd_hbm.at[pl.ds(bbase, bw)], bsig_v)
        for k in range(kb):
            pltpu.async_copy(ss_hbm.at[bidx_v.at[k]], bgth_v.at[k], sem).wait()

        acc_f = jnp.zeros((lanes,), jnp.float32)
        for c in range(fw // lanes):
            k, off = divmod(c * lanes, _IW)
            gth = fgth_v[k, pl.ds(off, lanes)]
            sl = pl.ds(c * lanes, lanes)
            acc_f = acc_f + fdat_v[sl] * (fsig_v[sl] + gth)

        acc_b = jnp.zeros((lanes,), jnp.float32)
        for c in range(bw // lanes):
            k, off = divmod(c * lanes, _IW)
            gth = bgth_v[k, pl.ds(off, lanes)]
            sl = pl.ds(c * lanes, lanes)
            acc_b = acc_b + bdat_v[sl] * (bsig_v[sl] + gth)

        acc_v[...] = acc_f * f_scale + acc_b * b_scale
        pltpu.sync_copy(acc_v, out_hbm.at[wid])

    return sc_fn


def _finalize_body(p_ref, o_ref):
    o_ref[...] = jnp.full((1, 1), jnp.sum(p_ref[...]), jnp.float32)


def _finalize(parts, *, interpret=False):
    return pl.pallas_call(
        _finalize_body,
        out_shape=jax.ShapeDtypeStruct((1, 1), jnp.float32),
        interpret=interpret,
    )(parts)


def kernel(pc_src, pc_dst, sigma_src, sigma_dst):
    b, _, m = pc_src.shape
    n = pc_dst.shape[2]
    src_t = jnp.transpose(pc_src, (0, 2, 1))
    dist_f, idx_f, dist_b, idx_b = _cdist_call(src_t, pc_dst)
    sc_fn = _make_sc_gather(b, m, n)
    parts = sc_fn(dist_f.reshape(-1), idx_f.reshape(-1, _IW),
                  sigma_src.reshape(-1), dist_b.reshape(-1),
                  idx_b.reshape(-1, _IW), sigma_dst.reshape(-1))
    return _finalize(parts)[0, 0]


# speedup vs baseline: 2.1760x; 1.0540x over previous
import functools

import jax
import jax.numpy as jnp
from jax import lax
from jax.experimental import pallas as pl
from jax.experimental.pallas import tpu as pltpu
from jax.experimental.pallas import tpu_sc as plsc

_NT = 2048


def _cdist_body(src_ref, dst_ref, dist_f_ref, idx_f_ref, dist_b_ref,
                idx_b_ref, row_key_ref):
    j = pl.program_id(1)
    nj = pl.num_programs(1)
    bb = pl.program_id(0)
    xs = src_ref[0]
    ys = dst_ref[0]
    m = xs.shape[0]
    nt = ys.shape[1]
    n_total = nj * nt
    f32, i32 = jnp.float32, jnp.int32

    d0 = xs[:, 0:1] - ys[0:1, :]
    d1 = xs[:, 1:2] - ys[1:2, :]
    d2_ = xs[:, 2:3] - ys[2:3, :]
    s = d0 * d0 + d1 * d1 + d2_ * d2_

    bits = lax.bitcast_convert_type(s, i32)
    km = bits & i32(-2048)

    iota_c = lax.broadcasted_iota(i32, (m, nt), 1)
    keyr = lax.bitcast_convert_type(km | iota_c, f32)
    tkey = lax.bitcast_convert_type(
        jnp.min(keyr, axis=1, keepdims=True), i32)
    gkey = tkey + j * nt

    @pl.when(j == 0)
    def _():
        row_key_ref[...] = gkey

    @pl.when(j > 0)
    def _():
        row_key_ref[...] = jnp.minimum(row_key_ref[...], gkey)

    iota_r = lax.broadcasted_iota(i32, (m, nt), 0)
    keyc = lax.bitcast_convert_type(km | iota_r, f32)
    ckey = lax.bitcast_convert_type(
        jnp.min(keyc, axis=0, keepdims=True), i32)
    cval = lax.bitcast_convert_type(ckey & i32(-2048), f32)
    dist_b_ref[...] = jnp.sqrt(jnp.maximum(cval, 0.0))[None]
    idx_b_ref[...] = ((ckey & 0x7FF) + bb * m)[None]

    @pl.when(j == nj - 1)
    def _():
        rk = row_key_ref[...]
        rval = lax.bitcast_convert_type(rk & i32(-2048), f32)
        dist_f_ref[...] = jnp.sqrt(jnp.maximum(rval, 0.0))[None]
        idx_f_ref[...] = ((rk & 0x7FF) + bb * n_total)[None]


def _cdist_call(src_t, dst, *, interpret=False):
    b, m, k = src_t.shape
    n = dst.shape[2]
    nj = n // _NT
    assert n <= 2048 and m <= 2048
    return pl.pallas_call(
        _cdist_body,
        grid=(b, nj),
        in_specs=[
            pl.BlockSpec((1, m, k), lambda b_, j: (b_, 0, 0)),
            pl.BlockSpec((1, k, _NT), lambda b_, j: (b_, 0, j)),
        ],
        out_specs=[
            pl.BlockSpec((1, m, 1), lambda b_, j: (b_, 0, 0)),
            pl.BlockSpec((1, m, 1), lambda b_, j: (b_, 0, 0)),
            pl.BlockSpec((1, 1, _NT), lambda b_, j: (b_, 0, j)),
            pl.BlockSpec((1, 1, _NT), lambda b_, j: (b_, 0, j)),
        ],
        out_shape=[
            jax.ShapeDtypeStruct((b, m, 1), jnp.float32),
            jax.ShapeDtypeStruct((b, m, 1), jnp.int32),
            jax.ShapeDtypeStruct((b, 1, n), jnp.float32),
            jax.ShapeDtypeStruct((b, 1, n), jnp.int32),
        ],
        scratch_shapes=[
            pltpu.VMEM((m, 1), jnp.int32),
        ],
        interpret=interpret,
    )(src_t, dst)


_IW = 128


def _make_sc_gather(b, m, n):
    info = plsc.get_sparse_core_info()
    nc, ns, lanes = info.num_cores, info.num_subcores, info.num_lanes
    nw = nc * ns
    fw = (b * m) // nw
    bw = (b * n) // nw
    assert (b * m) % (nw * _IW) == 0 and (b * n) % (nw * _IW) == 0
    kf = fw // _IW
    kb = bw // _IW
    f_scale = 0.5 / (b * m)
    b_scale = 0.5 / (b * n)
    mesh = plsc.VectorSubcoreMesh(core_axis_name="c", subcore_axis_name="s")

    @functools.partial(
        pl.kernel, mesh=mesh,
        out_type=jax.ShapeDtypeStruct((nw, lanes), jnp.float32),
        scratch_types=[
            pltpu.VMEM((fw,), jnp.int32),
            pltpu.VMEM((fw,), jnp.float32),
            pltpu.VMEM((fw,), jnp.float32),
            pltpu.VMEM((fw,), jnp.float32),
            pltpu.VMEM((bw,), jnp.int32),
            pltpu.VMEM((bw,), jnp.float32),
            pltpu.VMEM((bw,), jnp.float32),
            pltpu.VMEM((bw,), jnp.float32),
            pltpu.VMEM((lanes,), jnp.float32),
            pltpu.SemaphoreType.DMA,
            pltpu.SemaphoreType.DMA,
            pltpu.SemaphoreType.DMA,
        ],
    )
    def sc_fn(df_hbm, if_hbm, ss_hbm, db_hbm, ib_hbm, sd_hbm, out_hbm,
              fidx_v, fgth_v, fdat_v, fsig_v,
              bidx_v, bgth_v, bdat_v, bsig_v, acc_v,
              sem_i, sem_d, sem_g):
        wid = lax.axis_index("s") * nc + lax.axis_index("c")
        fbase = wid * fw
        bbase = wid * bw

        ci_f = pltpu.async_copy(if_hbm.at[pl.ds(fbase, fw)], fidx_v, sem_i)
        ci_b = pltpu.async_copy(ib_hbm.at[pl.ds(bbase, bw)], bidx_v, sem_i)
        cd = [
            pltpu.async_copy(df_hbm.at[pl.ds(fbase, fw)], fdat_v, sem_d),
            pltpu.async_copy(ss_hbm.at[pl.ds(fbase, fw)], fsig_v, sem_d),
            pltpu.async_copy(db_hbm.at[pl.ds(bbase, bw)], bdat_v, sem_d),
            pltpu.async_copy(sd_hbm.at[pl.ds(bbase, bw)], bsig_v, sem_d),
        ]
        ci_f.wait()
        ci_b.wait()
        cg = []
        for k in range(kf):
            sl = pl.ds(k * _IW, _IW)
            cg.append(pltpu.async_copy(sd_hbm.at[fidx_v.at[sl]],
                                       fgth_v.at[sl], sem_g))
        for k in range(kb):
            sl = pl.ds(k * _IW, _IW)
            cg.append(pltpu.async_copy(ss_hbm.at[bidx_v.at[sl]],
                                       bgth_v.at[sl], sem_g))
        for c in cd:
            c.wait()
        for c in cg:
            c.wait()

        acc_f = jnp.zeros((lanes,), jnp.float32)
        for c in range(fw // lanes):
            sl = pl.ds(c * lanes, lanes)
            acc_f = acc_f + fdat_v[sl] * (fsig_v[sl] + fgth_v[sl])

        acc_b = jnp.zeros((lanes,), jnp.float32)
        for c in range(bw // lanes):
            sl = pl.ds(c * lanes, lanes)
            acc_b = acc_b + bdat_v[sl] * (bsig_v[sl] + bgth_v[sl])

        acc_v[...] = acc_f * f_scale + acc_b * b_scale
        pltpu.sync_copy(acc_v, out_hbm.at[wid])

    return sc_fn


def _finalize_body(p_ref, o_ref):
    o_ref[...] = jnp.full((1, 1), jnp.sum(p_ref[...]), jnp.float32)


def _finalize(parts, *, interpret=False):
    return pl.pallas_call(
        _finalize_body,
        out_shape=jax.ShapeDtypeStruct((1, 1), jnp.float32),
        interpret=interpret,
    )(parts)


def kernel(pc_src, pc_dst, sigma_src, sigma_dst):
    b, _, m = pc_src.shape
    n = pc_dst.shape[2]
    src_t = jnp.transpose(pc_src, (0, 2, 1))
    dist_f, idx_f, dist_b, idx_b = _cdist_call(src_t, pc_dst)
    sc_fn = _make_sc_gather(b, m, n)
    parts = sc_fn(dist_f.reshape(-1), idx_f.reshape(-1),
                  sigma_src.reshape(-1), dist_b.reshape(-1),
                  idx_b.reshape(-1), sigma_dst.reshape(-1))
    return _finalize(parts)[0, 0]
